# baseline (speedup 1.0000x reference)
import functools

import jax
import jax.numpy as jnp
from jax import lax
from jax.experimental import pallas as pl
from jax.experimental.pallas import tpu as pltpu
from jax.experimental.pallas import tpu_sc as plsc

B = 16384
F = 26
V = 1000000

NC, NS, L = 2, 16, 16
NW = NC * NS
RPW = B // NW
EPW = RPW * F


GROUP = 208
KPG = GROUP // L
NG = EPW // GROUP


def _body(x_hbm, tab_hbm, out_hbm, idx_v, g_v, o_v, offs_v, sem):
    wid = lax.axis_index("s") * NC + lax.axis_index("c")
    pltpu.sync_copy(x_hbm.at[pl.ds(wid * EPW, EPW)], idx_v)

    lane = lax.iota(jnp.int32, L)
    for k in range(KPG):
        pos = lane + k * L
        f = pos - (k * L // F) * F
        f = jnp.where(f >= F, f - F, f)
        offs_v[pl.ds(k * L, L)] = f * V

    def idx_body(g, _):
        base = g * GROUP
        for k in range(KPG):
            sl = pl.ds(base + k * L, L)
            idx_v[sl] = idx_v[sl] + offs_v[pl.ds(k * L, L)]
        return 0

    lax.fori_loop(0, NG, idx_body, 0)

    pltpu.async_copy(tab_hbm.at[idx_v], g_v, sem).wait()

    def red_body(j, _):
        p = (lane + j * L) * F
        acc = plsc.load_gather(g_v, [p])
        for f in range(1, F):
            acc = acc + plsc.load_gather(g_v, [p + f])
        o_v[pl.ds(j * L, L)] = acc
        return 0

    lax.fori_loop(0, RPW // L, red_body, 0)

    pltpu.sync_copy(o_v, out_hbm.at[pl.ds(wid * RPW, RPW)])


@jax.jit
def kernel(X, tables):
    x_flat = X.reshape(B * F)
    tab_flat = tables.reshape(F * V)
    run = functools.partial(
        pl.kernel,
        out_type=jax.ShapeDtypeStruct((B,), jnp.float32),
        mesh=plsc.VectorSubcoreMesh(core_axis_name="c", subcore_axis_name="s"),
        scratch_types=[
            pltpu.VMEM((EPW,), jnp.int32),
            pltpu.VMEM((EPW,), jnp.float32),
            pltpu.VMEM((RPW,), jnp.float32),
            pltpu.VMEM((GROUP,), jnp.int32),
            pltpu.SemaphoreType.DMA,
        ],
        compiler_params=pltpu.CompilerParams(needs_layout_passes=False),
    )(_body)
    out = run(x_flat, tab_flat)
    return out.reshape(B, 1)


# speedup vs baseline: 1.0013x; 1.0013x over previous
import functools

import jax
import jax.numpy as jnp
from jax import lax
from jax.experimental import pallas as pl
from jax.experimental.pallas import tpu as pltpu
from jax.experimental.pallas import tpu_sc as plsc

B = 16384
F = 26
V = 1000000

NC, NS, L = 2, 16, 16
NW = NC * NS
RPW = B // NW
EPW = RPW * F


GROUP = 208
KPG = GROUP // L
NG = EPW // GROUP


def _body(x_hbm, tab_hbm, out_hbm, idx_v, g_v, o_v, offs_v, sem):
    wid = lax.axis_index("s") * NC + lax.axis_index("c")
    with jax.named_scope("stage_x"):
        pltpu.sync_copy(x_hbm.at[pl.ds(wid * EPW, EPW)], idx_v)

    lane = lax.iota(jnp.int32, L)
    for k in range(KPG):
        pos = lane + k * L
        f = pos - (k * L // F) * F
        f = jnp.where(f >= F, f - F, f)
        offs_v[pl.ds(k * L, L)] = f * V

    with jax.named_scope("idx_compute"):
        def idx_body(g, _):
            base = g * GROUP
            for k in range(KPG):
                sl = pl.ds(base + k * L, L)
                idx_v[sl] = idx_v[sl] + offs_v[pl.ds(k * L, L)]
            return 0

        lax.fori_loop(0, NG, idx_body, 0)

    with jax.named_scope("gather"):
        pltpu.async_copy(tab_hbm.at[idx_v], g_v, sem).wait()

    with jax.named_scope("reduce"):
        def red_body(j, _):
            p = (lane + j * L) * F
            acc = plsc.load_gather(g_v, [p])
            for f in range(1, F):
                acc = acc + plsc.load_gather(g_v, [p + f])
            o_v[pl.ds(j * L, L)] = acc
            return 0

        lax.fori_loop(0, RPW // L, red_body, 0)

    with jax.named_scope("writeout"):
        pltpu.sync_copy(o_v, out_hbm.at[pl.ds(wid * RPW, RPW)])


@jax.jit
def kernel(X, tables):
    x_flat = X.reshape(B * F)
    tab_flat = tables.reshape(F * V)
    run = functools.partial(
        pl.kernel,
        out_type=jax.ShapeDtypeStruct((B,), jnp.float32),
        mesh=plsc.VectorSubcoreMesh(core_axis_name="c", subcore_axis_name="s"),
        scratch_types=[
            pltpu.VMEM((EPW,), jnp.int32),
            pltpu.VMEM((EPW,), jnp.float32),
            pltpu.VMEM((RPW,), jnp.float32),
            pltpu.VMEM((GROUP,), jnp.int32),
            pltpu.SemaphoreType.DMA,
        ],
        compiler_params=pltpu.CompilerParams(needs_layout_passes=False),
    )(_body)
    out = run(x_flat, tab_flat)
    return out.reshape(B, 1)


# speedup vs baseline: 3.4426x; 3.4379x over previous
import functools

import jax
import jax.numpy as jnp
from jax import lax
from jax.experimental import pallas as pl
from jax.experimental.pallas import tpu as pltpu
from jax.experimental.pallas import tpu_sc as plsc

B = 16384
F = 26
V = 1000000

NC, NS, L = 2, 16, 16
NW = NC * NS
RPW = B // NW


def _body(x_hbm, *rest):
    tabs = rest[:F]
    out_hbm = rest[F]
    scratch = rest[F + 1:]
    xrows = scratch[0]
    xcols = scratch[1:1 + F]
    gbufs = scratch[1 + F:1 + 2 * F]
    o_v = scratch[1 + 2 * F]
    sem = scratch[2 + 2 * F]
    wid = lax.axis_index("s") * NC + lax.axis_index("c")
    base = wid * RPW

    pltpu.sync_copy(x_hbm.at[pl.ds(base, RPW)], xrows)
    lane = lax.iota(jnp.int32, L)

    def tr_body(j, _):
        rows = lane + j * L
        for f in range(F):
            xcols[f][pl.ds(j * L, L)] = plsc.load_gather(
                xrows, [rows, jnp.full((L,), f, jnp.int32)])
        return 0

    lax.fori_loop(0, RPW // L, tr_body, 0)

    copies = [
        pltpu.async_copy(tabs[f].at[xcols[f]], gbufs[f], sem)
        for f in range(F)
    ]
    for c in copies:
        c.wait()

    def red_body(j, _):
        sl = pl.ds(j * L, L)
        acc = gbufs[0][sl]
        for f in range(1, F):
            acc = acc + gbufs[f][sl]
        o_v[sl] = acc
        return 0

    lax.fori_loop(0, RPW // L, red_body, 0)

    pltpu.sync_copy(o_v, out_hbm.at[pl.ds(base, RPW)])


@jax.jit
def kernel(X, tables):
    tabs = tuple(tables[f] for f in range(F))
    run = functools.partial(
        pl.kernel,
        out_type=jax.ShapeDtypeStruct((B,), jnp.float32),
        mesh=plsc.VectorSubcoreMesh(core_axis_name="c", subcore_axis_name="s"),
        scratch_types=(
            [pltpu.VMEM((RPW, F), jnp.int32)]
            + [pltpu.VMEM((RPW,), jnp.int32)] * F
            + [pltpu.VMEM((RPW,), jnp.float32)] * F
            + [pltpu.VMEM((RPW,), jnp.float32)]
            + [pltpu.SemaphoreType.DMA]
        ),
        compiler_params=pltpu.CompilerParams(needs_layout_passes=False),
    )(_body)
    out = run(X, *tabs)
    return out.reshape(B, 1)


# speedup vs baseline: 16.3738x; 4.7562x over previous
import functools

import jax
import jax.numpy as jnp
from jax import lax
from jax.experimental import pallas as pl
from jax.experimental.pallas import tpu as pltpu
from jax.experimental.pallas import tpu_sc as plsc

B = 16384
F = 26
V = 1000000

NC, NS, L = 2, 16, 16
NW = NC * NS
RPW = B // NW
VP = 1000064


VMAIN = (V // 128) * 128
VTAIL = V - VMAIN


def _detile_body(tab_hbm, out_hbm,
                 buf0, buf1, tail0, tail1,
                 sem_in0, sem_in1, sem_out0, sem_out1, sem_t0, sem_t1):
    f = pl.program_id(0)

    def step(cur_buf, cur_tail, cur_si, cur_so, cur_st,
             oth_buf, oth_si, oth_so, oth_st, oth_tail):
        @pl.when(f == 0)
        def _():
            pltpu.make_async_copy(tab_hbm.at[0], cur_buf, cur_si).start()

        @pl.when(f >= 1)
        def _():
            pltpu.make_async_copy(
                oth_buf.at[pl.ds(0, VMAIN)],
                out_hbm.at[pl.ds((f - 1) * VP, VMAIN)], oth_so,
            ).wait()
            pltpu.make_async_copy(
                oth_tail, out_hbm.at[pl.ds((f - 1) * VP + VMAIN, 128)], oth_st,
            ).wait()

        @pl.when(f + 1 < F)
        def _():
            pltpu.make_async_copy(tab_hbm.at[f + 1], oth_buf, oth_si).start()

        pltpu.make_async_copy(tab_hbm.at[f], cur_buf, cur_si).wait()
        cur_tail[pl.ds(0, VTAIL)] = cur_buf[pl.ds(VMAIN, VTAIL)]
        pltpu.make_async_copy(
            cur_buf.at[pl.ds(0, VMAIN)], out_hbm.at[pl.ds(f * VP, VMAIN)],
            cur_so,
        ).start()
        pltpu.make_async_copy(
            cur_tail, out_hbm.at[pl.ds(f * VP + VMAIN, 128)], cur_st,
        ).start()

        @pl.when(f == F - 1)
        def _():
            pltpu.make_async_copy(
                cur_buf.at[pl.ds(0, VMAIN)],
                out_hbm.at[pl.ds((F - 1) * VP, VMAIN)], cur_so,
            ).wait()
            pltpu.make_async_copy(
                cur_tail, out_hbm.at[pl.ds((F - 1) * VP + VMAIN, 128)], cur_st,
            ).wait()

    @pl.when(f % 2 == 0)
    def _():
        step(buf0, tail0, sem_in0, sem_out0, sem_t0,
             buf1, sem_in1, sem_out1, sem_t1, tail1)

    @pl.when(f % 2 == 1)
    def _():
        step(buf1, tail1, sem_in1, sem_out1, sem_t1,
             buf0, sem_in0, sem_out0, sem_t0, tail0)


def _detile(tables):
    return pl.pallas_call(
        _detile_body,
        grid=(F,),
        in_specs=[pl.BlockSpec(memory_space=pl.ANY)],
        out_specs=pl.BlockSpec(memory_space=pl.ANY),
        out_shape=jax.ShapeDtypeStruct((F * VP,), jnp.float32),
        scratch_shapes=[
            pltpu.VMEM((V,), jnp.float32),
            pltpu.VMEM((V,), jnp.float32),
            pltpu.VMEM((128,), jnp.float32),
            pltpu.VMEM((128,), jnp.float32),
            pltpu.SemaphoreType.DMA,
            pltpu.SemaphoreType.DMA,
            pltpu.SemaphoreType.DMA,
            pltpu.SemaphoreType.DMA,
            pltpu.SemaphoreType.DMA,
            pltpu.SemaphoreType.DMA,
        ],
    )(tables)


def _body(x_hbm, tab_hbm, out_hbm, *scratch):
    xrows = scratch[0]
    xcols = scratch[1:1 + F]
    gbufs = scratch[1 + F:1 + 2 * F]
    o_v = scratch[1 + 2 * F]
    sem = scratch[2 + 2 * F]
    wid = lax.axis_index("s") * NC + lax.axis_index("c")
    base = wid * RPW

    pltpu.sync_copy(x_hbm.at[pl.ds(base, RPW)], xrows)
    lane = lax.iota(jnp.int32, L)

    def tr_body(j, _):
        rows = lane + j * L
        for f in range(F):
            xcols[f][pl.ds(j * L, L)] = plsc.load_gather(
                xrows, [rows, jnp.full((L,), f, jnp.int32)]) + f * VP
        return 0

    lax.fori_loop(0, RPW // L, tr_body, 0)

    copies = [
        pltpu.async_copy(tab_hbm.at[xcols[f]], gbufs[f], sem)
        for f in range(F)
    ]
    for c in copies:
        c.wait()

    def red_body(j, _):
        sl = pl.ds(j * L, L)
        acc = gbufs[0][sl]
        for f in range(1, F):
            acc = acc + gbufs[f][sl]
        o_v[sl] = acc
        return 0

    lax.fori_loop(0, RPW // L, red_body, 0)

    pltpu.sync_copy(o_v, out_hbm.at[pl.ds(base, RPW)])


@jax.jit
def kernel(X, tables):
    tab_flat = _detile(tables)
    run = functools.partial(
        pl.kernel,
        out_type=jax.ShapeDtypeStruct((B,), jnp.float32),
        mesh=plsc.VectorSubcoreMesh(core_axis_name="c", subcore_axis_name="s"),
        scratch_types=(
            [pltpu.VMEM((RPW, F), jnp.int32)]
            + [pltpu.VMEM((RPW,), jnp.int32)] * F
            + [pltpu.VMEM((RPW,), jnp.float32)] * F
            + [pltpu.VMEM((RPW,), jnp.float32)]
            + [pltpu.SemaphoreType.DMA]
        ),
        compiler_params=pltpu.CompilerParams(needs_layout_passes=False),
    )(_body)
    out = run(X, tab_flat)
    return out.reshape(B, 1)


# speedup vs baseline: 17.8680x; 1.0913x over previous
import functools

import jax
import jax.numpy as jnp
from jax import lax
from jax.experimental import pallas as pl
from jax.experimental.pallas import tpu as pltpu
from jax.experimental.pallas import tpu_sc as plsc

B = 16384
F = 26
V = 1000000

NC, NS, L = 2, 16, 16
NW = NC * NS
RPW = B // NW
VP = 1000064


VMAIN = (V // 128) * 128
VTAIL = V - VMAIN


def _detile_body(tab_hbm, out_hbm, buf_a, buf_b, tails_a, tails_b,
                 sem_in, sem_out, sem_tail):
    def in_group(g, buf):
        return pltpu.make_async_copy(tab_hbm.at[pl.ds(8 * g, 8)], buf, sem_in)

    def in_row(f, buf, r):
        return pltpu.make_async_copy(tab_hbm.at[f], buf.at[r], sem_in)

    def outs(buf, tails, rows, base_f):
        cps = []
        for r in rows:
            f = base_f + r
            tails[r, pl.ds(0, VTAIL)] = buf[r, pl.ds(VMAIN, VTAIL)]
            cps.append(pltpu.make_async_copy(
                buf.at[r, pl.ds(0, VMAIN)],
                out_hbm.at[pl.ds(f * VP, VMAIN)], sem_out))
            cps.append(pltpu.make_async_copy(
                tails.at[r], out_hbm.at[pl.ds(f * VP + VMAIN, 128)],
                sem_tail))
        for c in cps:
            c.start()
        return cps

    cp_a = in_group(0, buf_a)
    cp_b = in_group(1, buf_b)
    cp_a.start()
    cp_b.start()

    cp_a.wait()
    outs_a = outs(buf_a, tails_a, range(8), 0)
    cp_b.wait()
    outs_b = outs(buf_b, tails_b, range(8), 8)
    for c in outs_a:
        c.wait()
    cp_a = in_group(2, buf_a)
    cp_a.start()
    cp_a.wait()
    outs_a = outs(buf_a, tails_a, range(8), 16)
    for c in outs_b:
        c.wait()
    cp_b0 = in_row(24, buf_b, 0)
    cp_b1 = in_row(25, buf_b, 1)
    cp_b0.start()
    cp_b1.start()
    cp_b0.wait()
    cp_b1.wait()
    outs_b = outs(buf_b, tails_b, range(2), 24)
    for c in outs_a:
        c.wait()
    for c in outs_b:
        c.wait()


def _detile(tables):
    return pl.pallas_call(
        _detile_body,
        in_specs=[pl.BlockSpec(memory_space=pl.ANY)],
        out_specs=pl.BlockSpec(memory_space=pl.ANY),
        out_shape=jax.ShapeDtypeStruct((F * VP,), jnp.float32),
        scratch_shapes=[
            pltpu.VMEM((8, V), jnp.float32),
            pltpu.VMEM((8, V), jnp.float32),
            pltpu.VMEM((8, 128), jnp.float32),
            pltpu.VMEM((8, 128), jnp.float32),
            pltpu.SemaphoreType.DMA,
            pltpu.SemaphoreType.DMA,
            pltpu.SemaphoreType.DMA,
        ],
        compiler_params=pltpu.CompilerParams(
            vmem_limit_bytes=100 * 1024 * 1024),
    )(tables)


def _body(x_hbm, tab_hbm, out_hbm, *scratch):
    xrows = scratch[0]
    xcols = scratch[1:1 + F]
    gbufs = scratch[1 + F:1 + 2 * F]
    o_v = scratch[1 + 2 * F]
    sem = scratch[2 + 2 * F]
    wid = lax.axis_index("s") * NC + lax.axis_index("c")
    base = wid * RPW

    pltpu.sync_copy(x_hbm.at[pl.ds(base, RPW)], xrows)
    lane = lax.iota(jnp.int32, L)

    def tr_body(j, _):
        rows = lane + j * L
        for f in range(F):
            xcols[f][pl.ds(j * L, L)] = plsc.load_gather(
                xrows, [rows, jnp.full((L,), f, jnp.int32)]) + f * VP
        return 0

    lax.fori_loop(0, RPW // L, tr_body, 0)

    copies = [
        pltpu.async_copy(tab_hbm.at[xcols[f]], gbufs[f], sem)
        for f in range(F)
    ]
    for c in copies:
        c.wait()

    def red_body(j, _):
        sl = pl.ds(j * L, L)
        acc = gbufs[0][sl]
        for f in range(1, F):
            acc = acc + gbufs[f][sl]
        o_v[sl] = acc
        return 0

    lax.fori_loop(0, RPW // L, red_body, 0)

    pltpu.sync_copy(o_v, out_hbm.at[pl.ds(base, RPW)])


@jax.jit
def kernel(X, tables):
    tab_flat = _detile(tables)
    run = functools.partial(
        pl.kernel,
        out_type=jax.ShapeDtypeStruct((B,), jnp.float32),
        mesh=plsc.VectorSubcoreMesh(core_axis_name="c", subcore_axis_name="s"),
        scratch_types=(
            [pltpu.VMEM((RPW, F), jnp.int32)]
            + [pltpu.VMEM((RPW,), jnp.int32)] * F
            + [pltpu.VMEM((RPW,), jnp.float32)] * F
            + [pltpu.VMEM((RPW,), jnp.float32)]
            + [pltpu.SemaphoreType.DMA]
        ),
        compiler_params=pltpu.CompilerParams(needs_layout_passes=False),
    )(_body)
    out = run(X, tab_flat)
    return out.reshape(B, 1)
